# Initial kernel scaffold; baseline (speedup 1.0000x reference)
import jax
import jax.numpy as jnp
from jax.experimental import pallas as pl


def kernel(x, batch, edge_attr, edge_index, edge_batch, mean_x, mean_em):
    raise NotImplementedError("write your pallas kernel here")


# speedup vs baseline: 4.0100x; 4.0100x over previous
import jax
import jax.numpy as jnp
from jax import lax
from jax.experimental import pallas as pl
from jax.experimental.pallas import tpu as pltpu
from jax.experimental.pallas import tpu_sc as plsc

N = 100000
E = 6400000
B = 5000

NC = 2
NS = 16
NW = NC * NS

ROW = 128
ROWS_E = E // ROW
CH = 4
CHUNKS_E = ROWS_E // CH
CE = CH * ROW

N_PAD = 106496
NR = N_PAD // ROW
NTROW = NR // NS
RC = 4
NCHB = NTROW // RC
WSL = N_PAD // NS

B_PAD = 6144
BSEG = B_PAD // NS

NEG = jnp.float32(-3.0e38)


def _row_col(q):
  return [lax.shift_right_logical(q, 7), lax.bitwise_and(q, 127)]


def _edge_body(ea_h, dst_h, eb_h, w_o, es_o, ec_o,
               ea_v, d0, d1, d2, d3, b0, b1, b2, b3,
               wsrc, c0b, c1b, c2b, ones_v, zb, wstage, estage,
               wacc_s, es0_s, es1_s, es2_s, ec_s, sem):
  c = lax.axis_index("c")
  s = lax.axis_index("s")
  w = s * NC + c
  dstb = [d0, d1, d2, d3]
  ebb = [b0, b1, b2, b3]
  esb = [c0b, c1b, c2b]
  ess = [es0_s, es1_s, es2_s]

  iota = lax.iota(jnp.int32, 16)
  zf16 = jnp.zeros((16,), jnp.float32)
  of16 = jnp.ones((16,), jnp.float32)

  for g in range(512 // 16):
    zb[pl.ds(g * 16, 16)] = zf16
  for g in range(ROW // 16):
    ones_v[pl.ds(g * 16, 16)] = of16

  pltpu.sync_copy(zb.at[pl.ds(0, BSEG)], ec_s.at[pl.ds(s * BSEG, BSEG)])
  for t in range(3):
    pltpu.sync_copy(zb.at[pl.ds(0, BSEG)],
                    ess[t].at[pl.ds(s * BSEG, BSEG)])
  for k in range(WSL // 512):
    pltpu.sync_copy(zb, wacc_s.at[pl.ds(s * WSL + k * 512, 512)])
  plsc.subcore_barrier()

  c_lo = (w * CHUNKS_E) // NW
  c_hi = ((w + 1) * CHUNKS_E) // NW

  def chunk_body(ci, carry):
    cps = [pltpu.async_copy(ea_h.at[ci], ea_v, sem)]
    for j in range(CH):
      cps.append(pltpu.async_copy(
          dst_h.at[pl.ds(ci * CE + j * ROW, ROW)], dstb[j], sem))
      cps.append(pltpu.async_copy(
          eb_h.at[pl.ds(ci * CE + j * ROW, ROW)], ebb[j], sem))
    for cp in cps:
      cp.wait()

    for j in range(CH):
      for g in range(ROW // 16):
        k = j * ROW + g * 16
        q = (k + iota) * 3
        c0 = plsc.load_gather(ea_v, _row_col(q))
        c1 = plsc.load_gather(ea_v, _row_col(q + 1))
        c2 = plsc.load_gather(ea_v, _row_col(q + 2))
        sl = pl.ds(k, 16)
        c0b[sl] = c0
        c1b[sl] = c1
        c2b[sl] = c2
        wsrc[sl] = c0 + 2.0 * c1 + 3.0 * c2

    cps = []
    for j in range(CH):
      sl = pl.ds(j * ROW, ROW)
      cps.append(pltpu.async_copy(
          wsrc.at[sl], wacc_s.at[dstb[j]], sem, add=True))
      for t in range(3):
        cps.append(pltpu.async_copy(
            esb[t].at[sl], ess[t].at[ebb[j]], sem, add=True))
      cps.append(pltpu.async_copy(
          ones_v, ec_s.at[ebb[j]], sem, add=True))
    for cp in cps:
      cp.wait()
    return carry

  lax.fori_loop(c_lo, c_hi, chunk_body, 0)
  plsc.subcore_barrier()

  pltpu.sync_copy(wacc_s.at[pl.ds(s * WSL, WSL)], wstage)
  pltpu.sync_copy(wstage, w_o.at[pl.ds(c * N_PAD + s * WSL, WSL)])
  for t in range(3):
    pltpu.sync_copy(ess[t].at[pl.ds(s * BSEG, BSEG)], estage)
    pltpu.sync_copy(
        estage, es_o.at[pl.ds((c * 3 + t) * B_PAD + s * BSEG, BSEG)])
  pltpu.sync_copy(ec_s.at[pl.ds(s * BSEG, BSEG)], estage)
  pltpu.sync_copy(estage, ec_o.at[pl.ds(c * B_PAD + s * BSEG, BSEG)])


def _node_body(xq_h, bf_h, w_h, es_h, ec_h, mx_h, me_h, out_o,
               xq_v, nb0, nb1, nb2, nb3, px0, px1, px2, px3,
               w0_v, w1_v, ones_v, maxacc, zb, mxt, mvacc,
               xs0, xs1, xs2, xs3, cn_v,
               e00, e01, e02, e10, e11, e12, ec0, ec1,
               mxv, mev, out_v,
               mst_s, xa0_s, xa1_s, xa2_s, xa3_s, cnt_s, sem):
  c = lax.axis_index("c")
  s = lax.axis_index("s")
  bb = [nb0, nb1, nb2, nb3]
  pxb = [px0, px1, px2, px3]
  xsb = [xs0, xs1, xs2, xs3]
  xab = [xa0_s, xa1_s, xa2_s, xa3_s]
  esv = [[e00, e01, e02], [e10, e11, e12]]

  iota = lax.iota(jnp.int32, 16)
  zero16 = jnp.zeros((16,), jnp.int32)
  zf16 = jnp.zeros((16,), jnp.float32)
  of16 = jnp.ones((16,), jnp.float32)
  neg16 = jnp.full((16,), NEG, jnp.float32)
  lane0 = iota == 0
  tv4 = [jnp.full((16,), t, jnp.int32) for t in range(4)]

  for g in range(512 // 16):
    zb[pl.ds(g * 16, 16)] = zf16
  for g in range(ROW // 16):
    ones_v[pl.ds(g * 16, 16)] = of16
  for g in range(B_PAD // 16):
    plsc.store_scatter(maxacc, _row_col(g * 16 + iota), neg16)
  for t in range(4):
    pltpu.sync_copy(zb.at[pl.ds(0, BSEG)],
                    xab[t].at[pl.ds(s * BSEG, BSEG)])
  pltpu.sync_copy(zb.at[pl.ds(0, BSEG)], cnt_s.at[pl.ds(s * BSEG, BSEG)])

  pltpu.sync_copy(mx_h, mxv)
  pltpu.sync_copy(me_h, mev)
  mx = [plsc.load_gather(mxv, [zero16, tv4[t]]) for t in range(4)]
  me = [plsc.load_gather(mev, [zero16, tv4[t]]) for t in range(3)]
  plsc.subcore_barrier()

  def chunk_body(ch, carry):
    nrow0 = s * NTROW + ch * RC
    nbase = nrow0 * ROW
    cps = [
        pltpu.async_copy(xq_h.at[s * NCHB + ch], xq_v, sem),
        pltpu.async_copy(w_h.at[pl.ds(nbase, RC * ROW)], w0_v, sem),
        pltpu.async_copy(w_h.at[pl.ds(N_PAD + nbase, RC * ROW)], w1_v,
                         sem),
    ]
    for j in range(RC):
      cps.append(pltpu.async_copy(
          bf_h.at[pl.ds(nbase + j * ROW, ROW)], bb[j], sem))
    for cp in cps:
      cp.wait()

    for j in range(RC):
      for g in range(ROW // 16):
        k = j * ROW + g * 16
        sl = pl.ds(k, 16)
        q = (k + iota) * 4
        x0 = plsc.load_gather(xq_v, _row_col(q))
        x1 = plsc.load_gather(xq_v, _row_col(q + 1))
        x2 = plsc.load_gather(xq_v, _row_col(q + 2))
        x3 = plsc.load_gather(xq_v, _row_col(q + 3))
        px0[sl] = x0
        px1[sl] = x1
        px2[sl] = x2
        px3[sl] = x3
        bv = bb[j][pl.ds(g * 16, 16)]
        flow = (w0_v[sl] + w1_v[sl]
                - (4.0 * x0 + 3.0 * x1 + 2.0 * x2 + x3))
        bmin = jnp.min(bv)
        bmax = jnp.max(bv)

        def seg_body(d, carry):
          dv = jnp.full((16,), d, jnp.int32)
          mf = jnp.max(jnp.where(bv == d, flow, NEG))
          rc = _row_col(dv)
          cur = plsc.load_gather(maxacc, rc)
          plsc.store_scatter(
              maxacc, rc,
              jnp.maximum(cur, jnp.full((16,), mf, jnp.float32)),
              mask=lane0)
          return carry

        lax.fori_loop(bmin, bmax + 1, seg_body, 0)

    cps = []
    for j in range(RC):
      sl = pl.ds(j * ROW, ROW)
      for t in range(4):
        cps.append(pltpu.async_copy(
            pxb[t].at[sl], xab[t].at[bb[j]], sem, add=True))
      cps.append(pltpu.async_copy(
          ones_v, cnt_s.at[bb[j]], sem, add=True))
    for cp in cps:
      cp.wait()
    return carry

  lax.fori_loop(0, NCHB, chunk_body, 0)

  def merge_body(k, carry):
    pltpu.sync_copy(maxacc, mst_s.at[s])
    plsc.subcore_barrier()
    p = lax.bitwise_xor(s, lax.shift_left(1, k))
    pltpu.sync_copy(mst_s.at[p], mxt)
    plsc.subcore_barrier()
    for g in range(B_PAD // 16):
      rc = _row_col(g * 16 + iota)
      a = plsc.load_gather(maxacc, rc)
      b = plsc.load_gather(mxt, rc)
      plsc.store_scatter(maxacc, rc, jnp.maximum(a, b))
    return carry

  lax.fori_loop(0, 4, merge_body, 0)

  for g in range(BSEG // 16):
    mv = plsc.load_gather(maxacc, _row_col(s * BSEG + g * 16 + iota))
    mvacc[pl.ds(g * 16, 16)] = mv

  for t in range(4):
    pltpu.sync_copy(xab[t].at[pl.ds(s * BSEG, BSEG)], xsb[t])
  pltpu.sync_copy(cnt_s.at[pl.ds(s * BSEG, BSEG)], cn_v)
  for cc in range(NC):
    for t in range(3):
      pltpu.sync_copy(
          es_h.at[pl.ds((cc * 3 + t) * B_PAD + s * BSEG, BSEG)],
          esv[cc][t])
  pltpu.sync_copy(ec_h.at[pl.ds(s * BSEG, BSEG)], ec0)
  pltpu.sync_copy(ec_h.at[pl.ds(B_PAD + s * BSEG, BSEG)], ec1)

  for g in range(BSEG // 16):
    sl = pl.ds(g * 16, 16)
    xs = [xsb[t][sl] for t in range(4)]
    cx = cn_v[sl]
    es = [esv[0][t][sl] + esv[1][t][sl] for t in range(3)]
    ec = ec0[sl] + ec1[sl]
    mv = mvacc[sl]

    cxd = jnp.maximum(cx, 1.0)
    dx = [xs[t] / cxd - mx[t] for t in range(4)]
    l1 = (dx[0] * dx[0] + dx[1] * dx[1] + dx[2] * dx[2]
          + dx[3] * dx[3]) * 3.0
    ecd = jnp.maximum(ec, 1.0)
    de = [es[t] / ecd - me[t] for t in range(3)]
    l2 = (de[0] * de[0] + de[1] * de[1] + de[2] * de[2]) * 3.0
    z = ec - 21.0
    relu = jnp.where(z > 0.0, z, 0.3 * z)
    l3 = relu * relu
    l4 = jnp.maximum(mv, 0.0) * 4.0
    out_v[sl] = -(l1 + l2 + l3 + l4)

  @pl.when(c == 0)
  def _():
    pltpu.sync_copy(out_v, out_o.at[pl.ds(s * BSEG, BSEG)])


@jax.jit
def kernel(x, batch, edge_attr, edge_index, edge_batch, mean_x, mean_em):
  f32 = jnp.float32
  ea3 = edge_attr.reshape(CHUNKS_E, 12, ROW)
  dst1 = edge_index[1]
  eb1 = edge_batch

  mesh = plsc.VectorSubcoreMesh(core_axis_name="c", subcore_axis_name="s")
  cparams = pltpu.CompilerParams(needs_layout_passes=False)

  edge_phase = pl.kernel(
      _edge_body,
      out_type=[
          jax.ShapeDtypeStruct((NC * N_PAD,), f32),
          jax.ShapeDtypeStruct((NC * 3 * B_PAD,), f32),
          jax.ShapeDtypeStruct((NC * B_PAD,), f32),
      ],
      mesh=mesh,
      compiler_params=cparams,
      scratch_types=(
          [pltpu.VMEM((12, ROW), f32)]
          + [pltpu.VMEM((ROW,), jnp.int32) for _ in range(8)]
          + [pltpu.VMEM((CE,), f32) for _ in range(4)]
          + [
              pltpu.VMEM((ROW,), f32),
              pltpu.VMEM((512,), f32),
              pltpu.VMEM((WSL,), f32),
              pltpu.VMEM((BSEG,), f32),
              pltpu.VMEM_SHARED((N_PAD,), f32),
              pltpu.VMEM_SHARED((B_PAD,), f32),
              pltpu.VMEM_SHARED((B_PAD,), f32),
              pltpu.VMEM_SHARED((B_PAD,), f32),
              pltpu.VMEM_SHARED((B_PAD,), f32),
              pltpu.SemaphoreType.DMA,
          ]
      ),
  )
  w_o, es_o, ec_o = edge_phase(ea3, dst1, eb1)

  x_pad = jnp.concatenate([x, jnp.zeros((N_PAD - N, 4), f32)], axis=0)
  xq = x_pad.reshape(NS * NCHB, RC * 4, ROW)
  b_pad = jnp.concatenate([batch, jnp.full((N_PAD - N,), B, jnp.int32)])
  mxp = jnp.zeros((1, ROW), f32).at[0, :4].set(mean_x.reshape(4))
  mep = jnp.zeros((1, ROW), f32).at[0, :3].set(mean_em.reshape(3))

  node_phase = pl.kernel(
      _node_body,
      out_type=jax.ShapeDtypeStruct((B_PAD,), f32),
      mesh=mesh,
      compiler_params=cparams,
      scratch_types=(
          [pltpu.VMEM((RC * 4, ROW), f32)]
          + [pltpu.VMEM((ROW,), jnp.int32) for _ in range(RC)]
          + [pltpu.VMEM((RC * ROW,), f32) for _ in range(4)]
          + [
              pltpu.VMEM((RC * ROW,), f32),
              pltpu.VMEM((RC * ROW,), f32),
              pltpu.VMEM((ROW,), f32),
              pltpu.VMEM((48, ROW), f32),
              pltpu.VMEM((512,), f32),
              pltpu.VMEM((48, ROW), f32),
              pltpu.VMEM((BSEG,), f32),
          ]
          + [pltpu.VMEM((BSEG,), f32) for _ in range(5)]
          + [pltpu.VMEM((BSEG,), f32) for _ in range(8)]
          + [
              pltpu.VMEM((1, ROW), f32),
              pltpu.VMEM((1, ROW), f32),
              pltpu.VMEM((BSEG,), f32),
              pltpu.VMEM_SHARED((NS, 48, ROW), f32),
              pltpu.VMEM_SHARED((B_PAD,), f32),
              pltpu.VMEM_SHARED((B_PAD,), f32),
              pltpu.VMEM_SHARED((B_PAD,), f32),
              pltpu.VMEM_SHARED((B_PAD,), f32),
              pltpu.VMEM_SHARED((B_PAD,), f32),
              pltpu.SemaphoreType.DMA,
          ]
      ),
  )
  out = node_phase(xq, b_pad, w_o, es_o, ec_o, mxp, mep)
  return out[:B]
